# baseline (speedup 1.0000x reference)
import jax
import jax.numpy as jnp
from jax import lax
from jax.experimental import pallas as pl
from jax.experimental.pallas import tpu as pltpu
from jax.experimental.pallas import tpu_sc as plsc

N = 4096
E = 262144
C = 128
H = 4
HD = C // H
NN = N * N
NC = 2
NS = 16
NW = NC * NS
EPW = E // NW

_MESH = plsc.VectorSubcoreMesh(
    core_axis_name="c", subcore_axis_name="s", num_cores=NC, num_subcores=NS
)
_SC_PARAMS = pltpu.CompilerParams(needs_layout_passes=False)


def _zero_fill(ref, n):
  z = jnp.zeros((16,), jnp.float32)

  @pl.loop(0, n // 16)
  def _(i):
    ref[pl.ds(i * 16, 16)] = z


_SLAB = 256


def _sc_build_body(edge_ref, at_ref, deg_ref,
                   zbuf, sbuf, dbuf, idxb, didxb, onesb, degtab, slab, dsem):
  c = lax.axis_index("c")
  s = lax.axis_index("s")
  w = c * NS + s

  _zero_fill(zbuf, 16384)
  ones16 = jnp.full((16,), 1.0, jnp.float32)

  @pl.loop(0, 8)
  def _(i):
    onesb[pl.ds(i * 16, 16)] = ones16

  @pl.when(s == 0)
  def _():
    pltpu.sync_copy(zbuf.at[pl.ds(0, 2 * N)], degtab)

  pltpu.sync_copy(edge_ref.at[0, pl.ds(w * EPW, EPW)], sbuf)
  pltpu.sync_copy(edge_ref.at[1, pl.ds(w * EPW, EPW)], dbuf)

  plsc.subcore_barrier()

  @pl.loop(0, EPW // 1024)
  def _(g):
    base = g * 1024
    for r in range(8):
      for j in range(8):
        off = base + r * 128 + j * 16
        sv = sbuf[pl.ds(off, 16)]
        dv = dbuf[pl.ds(off, 16)]
        didxb[r, pl.ds(j * 16, 16)] = dv * 2
        didxb[8 + r, pl.ds(j * 16, 16)] = sv * 2 + 1
    descs = []
    for r in range(16):
      descs.append(
          pltpu.async_copy(onesb, degtab.at[didxb.at[r]], dsem, add=True))
    for d in descs:
      d.wait()

  plsc.subcore_barrier()

  @pl.when(s == 0)
  def _():
    pltpu.sync_copy(degtab, deg_ref.at[pl.ds(c * 2 * N, 2 * N)])

  @pl.loop(0, N // _SLAB)
  def _(p):
    r0 = p * _SLAB
    for i in range(4):
      pltpu.sync_copy(zbuf, slab.at[pl.ds((s * 16) * N + i * 16384, 16384)])

    @pl.when(s == 0)
    def _():
      pltpu.sync_copy(zbuf.at[pl.ds(0, N)], slab.at[pl.ds(_SLAB * N, N)])

    plsc.subcore_barrier()

    @pl.loop(0, EPW // 1024)
    def _(g):
      base = g * 1024
      for r in range(8):
        for j in range(8):
          off = base + r * 128 + j * 16
          sv = sbuf[pl.ds(off, 16)]
          dv = dbuf[pl.ds(off, 16)]
          inr = (sv >= r0) & (sv < r0 + _SLAB)
          idxb[r, pl.ds(j * 16, 16)] = jnp.where(
              inr, (sv - r0) * N + dv, _SLAB * N + dv)
      descs = []
      for r in range(8):
        descs.append(
            pltpu.async_copy(onesb, slab.at[idxb.at[r]], dsem, add=True))
      for d in descs:
        d.wait()

    plsc.subcore_barrier()
    for i in range(4):
      pltpu.sync_copy(
          slab.at[pl.ds((s * 16) * N + i * 16384, 16384)],
          at_ref.at[pl.ds(c * NN + (r0 + s * 16) * N + i * 16384, 16384)])


def _sc_build(edge_index):
  return pl.kernel(
      _sc_build_body,
      out_type=[
          jax.ShapeDtypeStruct((2 * NN,), jnp.float32),
          jax.ShapeDtypeStruct((2 * 2 * N,), jnp.float32),
      ],
      mesh=_MESH,
      scratch_types=[
          pltpu.VMEM((16384,), jnp.float32),
          pltpu.VMEM((EPW,), jnp.int32),
          pltpu.VMEM((EPW,), jnp.int32),
          pltpu.VMEM((8, 128), jnp.int32),
          pltpu.VMEM((16, 128), jnp.int32),
          pltpu.VMEM((128,), jnp.float32),
          pltpu.VMEM_SHARED((2 * N,), jnp.float32),
          pltpu.VMEM_SHARED(((_SLAB + 1) * N,), jnp.float32),
          pltpu.SemaphoreType.DMA,
      ],
      compiler_params=_SC_PARAMS,
  )(edge_index)


def _qkv_body(x_ref, wq_ref, bq_ref, wk_ref, bk_ref, wv_ref, bv_ref,
              q_ref, k_ref, v_ref):
  xx = x_ref[...]
  q_ref[...] = jnp.dot(xx, wq_ref[...],
                       preferred_element_type=jnp.float32) + bq_ref[...]
  k_ref[...] = jnp.dot(xx, wk_ref[...],
                       preferred_element_type=jnp.float32) + bk_ref[...]
  v_ref[...] = jnp.dot(xx, wv_ref[...],
                       preferred_element_type=jnp.float32) + bv_ref[...]


def _qkv(x, Wq, bq, Wk, bk, Wv, bv):
  return pl.pallas_call(
      _qkv_body,
      out_shape=[jax.ShapeDtypeStruct((N, C), jnp.float32)] * 3,
  )(x, Wq, bq.reshape(1, C), Wk, bk.reshape(1, C), Wv, bv.reshape(1, C))


def _atcombine_body(a_ref, o_ref):
  o_ref[...] = ((a_ref[0] + a_ref[1]) > 0.0).astype(jnp.bfloat16)


def _atcombine(at2):
  return pl.pallas_call(
      _atcombine_body,
      grid=(32,),
      in_specs=[pl.BlockSpec((2, 128, N), lambda i: (0, i, 0))],
      out_specs=pl.BlockSpec((128, N), lambda i: (i, 0)),
      out_shape=jax.ShapeDtypeStruct((N, N), jnp.bfloat16),
  )(at2)


_KP = 512


def _pow_dot(lhs_ref, atf_ref):
  def body(kp, acc):
    lsl = lhs_ref[:, pl.ds(kp * _KP, _KP)]
    panel = atf_ref[pl.ds(kp * _KP, _KP), :]
    return acc + lax.dot_general(lsl, panel, (((1,), (0,)), ((), ())),
                                 preferred_element_type=jnp.float32)

  init = jnp.zeros((_BR, N), jnp.float32)
  return lax.fori_loop(0, N // _KP, body, init)


def _dense_body(atf_ref, atb_ref, q_ref, k_ref, emb_ref, ex_ref, b2s_ref):
  a1 = atb_ref[...]
  p2 = _pow_dot(atb_ref, atf_ref)
  b2 = p2 > 0.0
  b2s_ref[...] = b2.astype(jnp.bfloat16)
  p3 = _pow_dot(b2s_ref, atf_ref)
  b3 = p3 > 0.0
  b1 = a1 > 0
  inv = jnp.float32(1.0 / (HD ** 0.5))
  for h in range(H):
    qh = q_ref[:, h * HD:(h + 1) * HD]
    kh = k_ref[:, h * HD:(h + 1) * HD]
    sc = lax.dot_general(qh, kh, (((1,), (1,)), ((), ())),
                         preferred_element_type=jnp.float32) * inv
    bias = jnp.where(b1, emb_ref[1, h],
                     jnp.where(b2, emb_ref[2, h],
                               jnp.where(b3, emb_ref[3, h], emb_ref[4, h])))
    ex_ref[h] = jnp.exp(sc + bias)


_BR = 64


def _dense_scores(atb, q, k, spd_emb):
  return pl.pallas_call(
      _dense_body,
      grid=(N // _BR,),
      in_specs=[
          pl.BlockSpec((N, N), lambda i: (0, 0)),
          pl.BlockSpec((_BR, N), lambda i: (i, 0)),
          pl.BlockSpec((_BR, C), lambda i: (i, 0)),
          pl.BlockSpec((N, C), lambda i: (0, 0)),
          pl.BlockSpec(memory_space=pltpu.SMEM),
      ],
      out_specs=pl.BlockSpec((H, _BR, N), lambda i: (0, i, 0)),
      out_shape=jax.ShapeDtypeStruct((H, N, N), jnp.float32),
      scratch_shapes=[pltpu.VMEM((_BR, N), jnp.bfloat16)],
      compiler_params=pltpu.CompilerParams(
          vmem_limit_bytes=60 * 1024 * 1024),
  )(atb, atb, q, k, spd_emb)


def _sc_edge_body(edge_ref, ex_ref, v_ref, seg_ref, agg_ref,
                  zbuf, zbuf1, sbuf, dbuf, gidx, sidx, vidx, aidx, exb, vrows,
                  vout, segtab, aggtab, gsem, vsem, ssem, asem):
  c = lax.axis_index("c")
  s = lax.axis_index("s")
  w = c * NS + s

  z16 = jnp.zeros((16,), jnp.float32)
  for jj in range(8):
    for rr in range(16):
      zbuf[rr, pl.ds(jj * 16, 16)] = z16
  _zero_fill(zbuf1, 2048)

  for r in range(16):
    pltpu.sync_copy(zbuf, aggtab.at[pl.ds(s * 256 + r * 16, 16)])

  @pl.when(s == 0)
  def _():
    for r in range(8):
      pltpu.sync_copy(zbuf1, segtab.at[pl.ds(r * 2048, 2048)])

  pltpu.sync_copy(edge_ref.at[0, pl.ds(w * EPW, EPW)], sbuf)
  pltpu.sync_copy(edge_ref.at[1, pl.ds(w * EPW, EPW)], dbuf)

  plsc.subcore_barrier()

  viota = lax.iota(jnp.int32, 16)

  @pl.loop(0, EPW // 256)
  def _(g):
    base = g * 256
    for r in range(2):
      for j in range(8):
        off = base + r * 128 + j * 16
        sv = sbuf[pl.ds(off, 16)]
        dv = dbuf[pl.ds(off, 16)]
        i0 = dv * N + sv
        s0 = dv * H
        for h in range(H):
          gidx[h * 2 + r, pl.ds(j * 16, 16)] = i0 + h * NN
          sidx[h * 2 + r, pl.ds(j * 16, 16)] = s0 + h
        vidx[r, pl.ds(j * 16, 16)] = sv
        aidx[r, pl.ds(j * 16, 16)] = dv
    gds = []
    for t in range(8):
      gds.append(pltpu.async_copy(ex_ref.at[gidx.at[t]],
                                  exb.at[pl.ds(t * 128, 128)], gsem))
    for r in range(2):
      gds.append(
          pltpu.async_copy(v_ref.at[vidx.at[r]],
                           vrows.at[pl.ds(r * 128, 128)], vsem))
    for d in gds:
      d.wait()
    sds = []
    for t in range(8):
      sds.append(
          pltpu.async_copy(exb.at[pl.ds(t * 128, 128)],
                           segtab.at[sidx.at[t]], ssem, add=True))
    @pl.loop(0, 16)
    def _(gg):
      rv = viota + gg * 16
      ebase = (gg // 8) * 128 + (gg % 8) * 16
      for h in range(H):
        wv = exb[pl.ds(h * 256 + ebase, 16)]
        for c in range(h * HD, (h + 1) * HD):
          cv = jnp.full((16,), c, jnp.int32)
          val = plsc.load_gather(vrows, [rv, cv])
          plsc.store_scatter(vout, [rv, cv], val * wv)
    for r in range(2):
      sds.append(
          pltpu.async_copy(vout.at[pl.ds(r * 128, 128)],
                           aggtab.at[aidx.at[r]], asem, add=True))
    for d in sds:
      d.wait()

  plsc.subcore_barrier()

  for r in range(16):
    pltpu.sync_copy(aggtab.at[pl.ds(s * 256 + r * 16, 16)],
                    agg_ref.at[pl.ds(c * N + s * 256 + r * 16, 16)])

  @pl.when(s == 0)
  def _():
    pltpu.sync_copy(segtab, seg_ref.at[pl.ds(c * N * H, N * H)])


def _sc_edge(edge_index, exf, v):
  return pl.kernel(
      _sc_edge_body,
      out_type=[
          jax.ShapeDtypeStruct((2 * N * H,), jnp.float32),
          jax.ShapeDtypeStruct((2 * N, C), jnp.float32),
      ],
      mesh=_MESH,
      scratch_types=[
          pltpu.VMEM((16, 128), jnp.float32),
          pltpu.VMEM((2048,), jnp.float32),
          pltpu.VMEM((EPW,), jnp.int32),
          pltpu.VMEM((EPW,), jnp.int32),
          pltpu.VMEM((8, 128), jnp.int32),
          pltpu.VMEM((8, 128), jnp.int32),
          pltpu.VMEM((2, 128), jnp.int32),
          pltpu.VMEM((2, 128), jnp.int32),
          pltpu.VMEM((1024,), jnp.float32),
          pltpu.VMEM((256, C), jnp.float32),
          pltpu.VMEM((256, C), jnp.float32),
          pltpu.VMEM_SHARED((N * H,), jnp.float32),
          pltpu.VMEM_SHARED((N, C), jnp.float32),
          pltpu.SemaphoreType.DMA,
          pltpu.SemaphoreType.DMA,
          pltpu.SemaphoreType.DMA,
          pltpu.SemaphoreType.DMA,
      ],
      compiler_params=_SC_PARAMS,
  )(edge_index, exf, v)


def _final_body(agg_ref, seg_ref, x_ref, deg_ref, g_ref, b_ref, y_ref):
  d2 = deg_ref[0] + deg_ref[1]
  dcol = jnp.sum(d2, axis=1, keepdims=True)
  segc = seg_ref[0] + seg_ref[1]
  segb = jnp.reshape(
      lax.broadcast_in_dim(segc, (N, H, HD), (0, 1)), (N, C))
  att = (agg_ref[0] + agg_ref[1]) / (segb + 1e-16)
  hh = att + x_ref[...] + dcol
  mu = jnp.mean(hh, axis=1, keepdims=True)
  dd = hh - mu
  var = jnp.mean(dd * dd, axis=1, keepdims=True)
  y_ref[...] = dd / jnp.sqrt(var + 1e-5) * g_ref[...] + b_ref[...]


def _final(agg, seg, x, degs, ln_g, ln_b):
  return pl.pallas_call(
      _final_body,
      out_shape=jax.ShapeDtypeStruct((N, C), jnp.float32),
  )(agg, seg, x, degs, ln_g.reshape(1, C), ln_b.reshape(1, C))


def kernel(x, edge_index, Wq, bq, Wk, bk, Wv, bv, spd_emb, ln_g, ln_b):
  at2_flat, degs_flat = _sc_build(edge_index)
  q, k, v = _qkv(x, Wq, bq, Wk, bk, Wv, bv)
  atb = _atcombine(at2_flat.reshape(2, N, N))
  ex = _dense_scores(atb, q, k, spd_emb)
  exf = ex.reshape(H * NN)
  seg, agg = _sc_edge(edge_index, exf, v)
  return _final(agg.reshape(2, N, C), seg.reshape(2, N, H), x,
                degs_flat.reshape(2, N, 2), ln_g, ln_b)


# speedup vs baseline: 1.0854x; 1.0854x over previous
import jax
import jax.numpy as jnp
from jax import lax
from jax.experimental import pallas as pl
from jax.experimental.pallas import tpu as pltpu
from jax.experimental.pallas import tpu_sc as plsc

N = 4096
E = 262144
C = 128
H = 4
HD = C // H
NN = N * N
NC = 2
NS = 16
NW = NC * NS
EPW = E // NW

_MESH = plsc.VectorSubcoreMesh(
    core_axis_name="c", subcore_axis_name="s", num_cores=NC, num_subcores=NS
)
_SC_PARAMS = pltpu.CompilerParams(needs_layout_passes=False)


def _zero_fill(ref, n):
  z = jnp.zeros((16,), jnp.float32)

  @pl.loop(0, n // 16)
  def _(i):
    ref[pl.ds(i * 16, 16)] = z


_SLAB = 256


def _sc_build_body(edge_ref, at_ref, deg_ref,
                   zbuf, sbuf, dbuf, idxb, didxb, onesb, degtab, slab, dsem):
  c = lax.axis_index("c")
  s = lax.axis_index("s")
  w = c * NS + s

  _zero_fill(zbuf, 16384)
  ones16 = jnp.full((16,), 1.0, jnp.float32)

  @pl.loop(0, 8)
  def _(i):
    onesb[pl.ds(i * 16, 16)] = ones16

  @pl.when(s == 0)
  def _():
    pltpu.sync_copy(zbuf.at[pl.ds(0, 2 * N)], degtab)

  pltpu.sync_copy(edge_ref.at[0, pl.ds(w * EPW, EPW)], sbuf)
  pltpu.sync_copy(edge_ref.at[1, pl.ds(w * EPW, EPW)], dbuf)

  plsc.subcore_barrier()

  @pl.loop(0, EPW // 1024)
  def _(g):
    base = g * 1024
    for r in range(8):
      for j in range(8):
        off = base + r * 128 + j * 16
        sv = sbuf[pl.ds(off, 16)]
        dv = dbuf[pl.ds(off, 16)]
        didxb[r, pl.ds(j * 16, 16)] = dv * 2
        didxb[8 + r, pl.ds(j * 16, 16)] = sv * 2 + 1
    descs = []
    for r in range(16):
      descs.append(
          pltpu.async_copy(onesb, degtab.at[didxb.at[r]], dsem, add=True))
    for d in descs:
      d.wait()

  plsc.subcore_barrier()

  @pl.when(s == 0)
  def _():
    pltpu.sync_copy(degtab, deg_ref.at[pl.ds(c * 2 * N, 2 * N)])

  @pl.loop(0, N // _SLAB)
  def _(p):
    r0 = p * _SLAB
    for i in range(4):
      pltpu.sync_copy(zbuf, slab.at[pl.ds((s * 16) * N + i * 16384, 16384)])

    @pl.when(s == 0)
    def _():
      pltpu.sync_copy(zbuf.at[pl.ds(0, N)], slab.at[pl.ds(_SLAB * N, N)])

    plsc.subcore_barrier()

    @pl.loop(0, EPW // 1024)
    def _(g):
      base = g * 1024
      for r in range(8):
        for j in range(8):
          off = base + r * 128 + j * 16
          sv = sbuf[pl.ds(off, 16)]
          dv = dbuf[pl.ds(off, 16)]
          inr = (sv >= r0) & (sv < r0 + _SLAB)
          idxb[r, pl.ds(j * 16, 16)] = jnp.where(
              inr, (sv - r0) * N + dv, _SLAB * N + dv)
      descs = []
      for r in range(8):
        descs.append(
            pltpu.async_copy(onesb, slab.at[idxb.at[r]], dsem, add=True))
      for d in descs:
        d.wait()

    plsc.subcore_barrier()
    for i in range(4):
      pltpu.sync_copy(
          slab.at[pl.ds((s * 16) * N + i * 16384, 16384)],
          at_ref.at[pl.ds(c * NN + (r0 + s * 16) * N + i * 16384, 16384)])


def _sc_build(edge_index):
  return pl.kernel(
      _sc_build_body,
      out_type=[
          jax.ShapeDtypeStruct((2 * NN,), jnp.float32),
          jax.ShapeDtypeStruct((2 * 2 * N,), jnp.float32),
      ],
      mesh=_MESH,
      scratch_types=[
          pltpu.VMEM((16384,), jnp.float32),
          pltpu.VMEM((EPW,), jnp.int32),
          pltpu.VMEM((EPW,), jnp.int32),
          pltpu.VMEM((8, 128), jnp.int32),
          pltpu.VMEM((16, 128), jnp.int32),
          pltpu.VMEM((128,), jnp.float32),
          pltpu.VMEM_SHARED((2 * N,), jnp.float32),
          pltpu.VMEM_SHARED(((_SLAB + 1) * N,), jnp.float32),
          pltpu.SemaphoreType.DMA,
      ],
      compiler_params=_SC_PARAMS,
  )(edge_index)


def _qkv_body(x_ref, wq_ref, bq_ref, wk_ref, bk_ref, wv_ref, bv_ref,
              q_ref, k_ref, v_ref):
  xx = x_ref[...]
  q_ref[...] = jnp.dot(xx, wq_ref[...],
                       preferred_element_type=jnp.float32) + bq_ref[...]
  k_ref[...] = jnp.dot(xx, wk_ref[...],
                       preferred_element_type=jnp.float32) + bk_ref[...]
  v_ref[...] = jnp.dot(xx, wv_ref[...],
                       preferred_element_type=jnp.float32) + bv_ref[...]


def _qkv(x, Wq, bq, Wk, bk, Wv, bv):
  return pl.pallas_call(
      _qkv_body,
      out_shape=[jax.ShapeDtypeStruct((N, C), jnp.float32)] * 3,
  )(x, Wq, bq.reshape(1, C), Wk, bk.reshape(1, C), Wv, bv.reshape(1, C))


def _atcombine_body(a_ref, o_ref):
  o_ref[...] = ((a_ref[0] + a_ref[1]) > 0.0).astype(jnp.bfloat16)


def _atcombine(at2):
  return pl.pallas_call(
      _atcombine_body,
      grid=(32,),
      in_specs=[pl.BlockSpec((2, 128, N), lambda i: (0, i, 0))],
      out_specs=pl.BlockSpec((128, N), lambda i: (i, 0)),
      out_shape=jax.ShapeDtypeStruct((N, N), jnp.bfloat16),
  )(at2)


_KP = 512


def _pow_dot(lhs_ref, atf_ref):
  def body(kp, acc):
    lsl = lhs_ref[:, pl.ds(kp * _KP, _KP)]
    panel = atf_ref[pl.ds(kp * _KP, _KP), :]
    return acc + lax.dot_general(lsl, panel, (((1,), (0,)), ((), ())),
                                 preferred_element_type=jnp.float32)

  init = jnp.zeros((_BR, N), jnp.float32)
  return lax.fori_loop(0, N // _KP, body, init)


def _dense_body(atf_ref, atb_ref, q_ref, k_ref, emb_ref, ex_ref, b2s_ref):
  a1 = atb_ref[...]
  p2 = _pow_dot(atb_ref, atf_ref)
  b2 = p2 > 0.0
  b2s_ref[...] = b2.astype(jnp.bfloat16)
  p3 = _pow_dot(b2s_ref, atf_ref)
  b3 = p3 > 0.0
  b1 = a1 > 0
  inv = jnp.float32(1.0 / (HD ** 0.5))
  for h in range(H):
    qh = q_ref[:, h * HD:(h + 1) * HD]
    kh = k_ref[:, h * HD:(h + 1) * HD]
    sc = lax.dot_general(qh, kh, (((1,), (1,)), ((), ())),
                         preferred_element_type=jnp.float32) * inv
    bias = jnp.where(b1, emb_ref[1, h],
                     jnp.where(b2, emb_ref[2, h],
                               jnp.where(b3, emb_ref[3, h], emb_ref[4, h])))
    ex_ref[h] = jnp.exp(sc + bias)


_BR = 64


def _dense_scores(atb, q, k, spd_emb):
  return pl.pallas_call(
      _dense_body,
      grid=(N // _BR,),
      in_specs=[
          pl.BlockSpec((N, N), lambda i: (0, 0)),
          pl.BlockSpec((_BR, N), lambda i: (i, 0)),
          pl.BlockSpec((_BR, C), lambda i: (i, 0)),
          pl.BlockSpec((N, C), lambda i: (0, 0)),
          pl.BlockSpec(memory_space=pltpu.SMEM),
      ],
      out_specs=pl.BlockSpec((H, _BR, N), lambda i: (0, i, 0)),
      out_shape=jax.ShapeDtypeStruct((H, N, N), jnp.float32),
      scratch_shapes=[pltpu.VMEM((_BR, N), jnp.bfloat16)],
      compiler_params=pltpu.CompilerParams(
          vmem_limit_bytes=60 * 1024 * 1024),
  )(atb, atb, q, k, spd_emb)


def _sc_edge_body(edge_ref, ex_ref, v_ref, seg_ref, agg_ref,
                  zbuf, zbuf1, sbuf, dbuf, gidx, sidx, vidx, aidx, exb, vrows,
                  vout, segtab, aggtab, gsem, vsem, ssem, asem):
  c = lax.axis_index("c")
  s = lax.axis_index("s")
  w = c * NS + s

  z16 = jnp.zeros((16,), jnp.float32)
  for jj in range(8):
    for rr in range(16):
      zbuf[rr, pl.ds(jj * 16, 16)] = z16
  _zero_fill(zbuf1, 2048)

  for r in range(16):
    pltpu.sync_copy(zbuf, aggtab.at[pl.ds(s * 256 + r * 16, 16)])

  @pl.when(s == 0)
  def _():
    for r in range(8):
      pltpu.sync_copy(zbuf1, segtab.at[pl.ds(r * 2048, 2048)])

  pltpu.sync_copy(edge_ref.at[0, pl.ds(w * EPW, EPW)], sbuf)
  pltpu.sync_copy(edge_ref.at[1, pl.ds(w * EPW, EPW)], dbuf)

  plsc.subcore_barrier()

  viota = lax.iota(jnp.int32, 16)

  @pl.loop(0, EPW // 256)
  def _(g):
    base = g * 256
    for r in range(2):
      for j in range(8):
        off = base + r * 128 + j * 16
        sv = sbuf[pl.ds(off, 16)]
        dv = dbuf[pl.ds(off, 16)]
        i0 = dv * N + sv
        s0 = dv * H
        for h in range(H):
          gidx[h * 2 + r, pl.ds(j * 16, 16)] = i0 + h * NN
          sidx[h * 2 + r, pl.ds(j * 16, 16)] = s0 + h
        vidx[r, pl.ds(j * 16, 16)] = sv
        aidx[r, pl.ds(j * 16, 16)] = dv
    gds = []
    for t in range(8):
      gds.append(pltpu.async_copy(ex_ref.at[gidx.at[t]],
                                  exb.at[pl.ds(t * 128, 128)], gsem))
    for r in range(2):
      gds.append(
          pltpu.async_copy(v_ref.at[vidx.at[r]],
                           vrows.at[pl.ds(r * 128, 128)], vsem))
    for d in gds:
      d.wait()
    sds = []
    for t in range(8):
      sds.append(
          pltpu.async_copy(exb.at[pl.ds(t * 128, 128)],
                           segtab.at[sidx.at[t]], ssem, add=True))
    @plsc.parallel_loop(0, 16, 1, unroll=2)
    def _(gg):
      rv = viota + gg * 16
      ebase = (gg // 8) * 128 + (gg % 8) * 16
      for h in range(H):
        wv = exb[pl.ds(h * 256 + ebase, 16)]
        for c in range(h * HD, (h + 1) * HD):
          cv = jnp.full((16,), c, jnp.int32)
          val = plsc.load_gather(vrows, [rv, cv])
          plsc.store_scatter(vout, [rv, cv], val * wv)
    for r in range(2):
      sds.append(
          pltpu.async_copy(vout.at[pl.ds(r * 128, 128)],
                           aggtab.at[aidx.at[r]], asem, add=True))
    for d in sds:
      d.wait()

  plsc.subcore_barrier()

  for r in range(16):
    pltpu.sync_copy(aggtab.at[pl.ds(s * 256 + r * 16, 16)],
                    agg_ref.at[pl.ds(c * N + s * 256 + r * 16, 16)])

  @pl.when(s == 0)
  def _():
    pltpu.sync_copy(segtab, seg_ref.at[pl.ds(c * N * H, N * H)])


def _sc_edge(edge_index, exf, v):
  return pl.kernel(
      _sc_edge_body,
      out_type=[
          jax.ShapeDtypeStruct((2 * N * H,), jnp.float32),
          jax.ShapeDtypeStruct((2 * N, C), jnp.float32),
      ],
      mesh=_MESH,
      scratch_types=[
          pltpu.VMEM((16, 128), jnp.float32),
          pltpu.VMEM((2048,), jnp.float32),
          pltpu.VMEM((EPW,), jnp.int32),
          pltpu.VMEM((EPW,), jnp.int32),
          pltpu.VMEM((8, 128), jnp.int32),
          pltpu.VMEM((8, 128), jnp.int32),
          pltpu.VMEM((2, 128), jnp.int32),
          pltpu.VMEM((2, 128), jnp.int32),
          pltpu.VMEM((1024,), jnp.float32),
          pltpu.VMEM((256, C), jnp.float32),
          pltpu.VMEM((256, C), jnp.float32),
          pltpu.VMEM_SHARED((N * H,), jnp.float32),
          pltpu.VMEM_SHARED((N, C), jnp.float32),
          pltpu.SemaphoreType.DMA,
          pltpu.SemaphoreType.DMA,
          pltpu.SemaphoreType.DMA,
          pltpu.SemaphoreType.DMA,
      ],
      compiler_params=_SC_PARAMS,
  )(edge_index, exf, v)


def _final_body(agg_ref, seg_ref, x_ref, deg_ref, g_ref, b_ref, y_ref):
  d2 = deg_ref[0] + deg_ref[1]
  dcol = jnp.sum(d2, axis=1, keepdims=True)
  segc = seg_ref[0] + seg_ref[1]
  segb = jnp.reshape(
      lax.broadcast_in_dim(segc, (N, H, HD), (0, 1)), (N, C))
  att = (agg_ref[0] + agg_ref[1]) / (segb + 1e-16)
  hh = att + x_ref[...] + dcol
  mu = jnp.mean(hh, axis=1, keepdims=True)
  dd = hh - mu
  var = jnp.mean(dd * dd, axis=1, keepdims=True)
  y_ref[...] = dd / jnp.sqrt(var + 1e-5) * g_ref[...] + b_ref[...]


def _final(agg, seg, x, degs, ln_g, ln_b):
  return pl.pallas_call(
      _final_body,
      out_shape=jax.ShapeDtypeStruct((N, C), jnp.float32),
  )(agg, seg, x, degs, ln_g.reshape(1, C), ln_b.reshape(1, C))


def kernel(x, edge_index, Wq, bq, Wk, bk, Wv, bv, spd_emb, ln_g, ln_b):
  at2_flat, degs_flat = _sc_build(edge_index)
  q, k, v = _qkv(x, Wq, bq, Wk, bk, Wv, bv)
  atb = _atcombine(at2_flat.reshape(2, N, N))
  ex = _dense_scores(atb, q, k, spd_emb)
  exf = ex.reshape(H * NN)
  seg, agg = _sc_edge(edge_index, exf, v)
  return _final(agg.reshape(2, N, C), seg.reshape(2, N, H), x,
                degs_flat.reshape(2, N, 2), ln_g, ln_b)


# speedup vs baseline: 1.0869x; 1.0014x over previous
import jax
import jax.numpy as jnp
from jax import lax
from jax.experimental import pallas as pl
from jax.experimental.pallas import tpu as pltpu
from jax.experimental.pallas import tpu_sc as plsc

N = 4096
E = 262144
C = 128
H = 4
HD = C // H
NN = N * N
NC = 2
NS = 16
NW = NC * NS
EPW = E // NW

_MESH = plsc.VectorSubcoreMesh(
    core_axis_name="c", subcore_axis_name="s", num_cores=NC, num_subcores=NS
)
_SC_PARAMS = pltpu.CompilerParams(needs_layout_passes=False)


def _zero_fill(ref, n):
  z = jnp.zeros((16,), jnp.float32)

  @pl.loop(0, n // 16)
  def _(i):
    ref[pl.ds(i * 16, 16)] = z


_SLAB = 256


def _sc_build_body(edge_ref, at_ref, deg_ref,
                   zbuf, sbuf, dbuf, idxb, didxb, onesb, degtab, slab, dsem):
  c = lax.axis_index("c")
  s = lax.axis_index("s")
  w = c * NS + s

  _zero_fill(zbuf, 16384)
  ones16 = jnp.full((16,), 1.0, jnp.float32)

  @pl.loop(0, 8)
  def _(i):
    onesb[pl.ds(i * 16, 16)] = ones16

  @pl.when(s == 0)
  def _():
    pltpu.sync_copy(zbuf.at[pl.ds(0, 2 * N)], degtab)

  pltpu.sync_copy(edge_ref.at[0, pl.ds(w * EPW, EPW)], sbuf)
  pltpu.sync_copy(edge_ref.at[1, pl.ds(w * EPW, EPW)], dbuf)

  plsc.subcore_barrier()

  @pl.loop(0, EPW // 1024)
  def _(g):
    base = g * 1024
    for r in range(8):
      for j in range(8):
        off = base + r * 128 + j * 16
        sv = sbuf[pl.ds(off, 16)]
        dv = dbuf[pl.ds(off, 16)]
        didxb[r, pl.ds(j * 16, 16)] = dv * 2
        didxb[8 + r, pl.ds(j * 16, 16)] = sv * 2 + 1
    descs = []
    for r in range(16):
      descs.append(
          pltpu.async_copy(onesb, degtab.at[didxb.at[r]], dsem, add=True))
    for d in descs:
      d.wait()

  plsc.subcore_barrier()

  @pl.when(s == 0)
  def _():
    pltpu.sync_copy(degtab, deg_ref.at[pl.ds(c * 2 * N, 2 * N)])

  @pl.loop(0, N // _SLAB)
  def _(p):
    r0 = p * _SLAB
    for i in range(4):
      pltpu.sync_copy(zbuf, slab.at[pl.ds((s * 16) * N + i * 16384, 16384)])

    @pl.when(s == 0)
    def _():
      pltpu.sync_copy(zbuf.at[pl.ds(0, N)], slab.at[pl.ds(_SLAB * N, N)])

    plsc.subcore_barrier()

    @pl.loop(0, EPW // 1024)
    def _(g):
      base = g * 1024
      for r in range(8):
        for j in range(8):
          off = base + r * 128 + j * 16
          sv = sbuf[pl.ds(off, 16)]
          dv = dbuf[pl.ds(off, 16)]
          inr = (sv >= r0) & (sv < r0 + _SLAB)
          idxb[r, pl.ds(j * 16, 16)] = jnp.where(
              inr, (sv - r0) * N + dv, _SLAB * N + dv)
      descs = []
      for r in range(8):
        descs.append(
            pltpu.async_copy(onesb, slab.at[idxb.at[r]], dsem, add=True))
      for d in descs:
        d.wait()

    plsc.subcore_barrier()
    for i in range(4):
      pltpu.sync_copy(
          slab.at[pl.ds((s * 16) * N + i * 16384, 16384)],
          at_ref.at[pl.ds(c * NN + (r0 + s * 16) * N + i * 16384, 16384)])


def _sc_build(edge_index):
  return pl.kernel(
      _sc_build_body,
      out_type=[
          jax.ShapeDtypeStruct((2 * NN,), jnp.float32),
          jax.ShapeDtypeStruct((2 * 2 * N,), jnp.float32),
      ],
      mesh=_MESH,
      scratch_types=[
          pltpu.VMEM((16384,), jnp.float32),
          pltpu.VMEM((EPW,), jnp.int32),
          pltpu.VMEM((EPW,), jnp.int32),
          pltpu.VMEM((8, 128), jnp.int32),
          pltpu.VMEM((16, 128), jnp.int32),
          pltpu.VMEM((128,), jnp.float32),
          pltpu.VMEM_SHARED((2 * N,), jnp.float32),
          pltpu.VMEM_SHARED(((_SLAB + 1) * N,), jnp.float32),
          pltpu.SemaphoreType.DMA,
      ],
      compiler_params=_SC_PARAMS,
  )(edge_index)


def _qkv_body(x_ref, wq_ref, bq_ref, wk_ref, bk_ref, wv_ref, bv_ref,
              q_ref, k_ref, v_ref):
  xx = x_ref[...]
  q_ref[...] = jnp.dot(xx, wq_ref[...],
                       preferred_element_type=jnp.float32) + bq_ref[...]
  k_ref[...] = jnp.dot(xx, wk_ref[...],
                       preferred_element_type=jnp.float32) + bk_ref[...]
  v_ref[...] = jnp.dot(xx, wv_ref[...],
                       preferred_element_type=jnp.float32) + bv_ref[...]


def _qkv(x, Wq, bq, Wk, bk, Wv, bv):
  return pl.pallas_call(
      _qkv_body,
      out_shape=[jax.ShapeDtypeStruct((N, C), jnp.float32)] * 3,
  )(x, Wq, bq.reshape(1, C), Wk, bk.reshape(1, C), Wv, bv.reshape(1, C))


def _atcombine_body(a_ref, o_ref):
  o_ref[...] = ((a_ref[0] + a_ref[1]) > 0.0).astype(jnp.bfloat16)


def _atcombine(at2):
  return pl.pallas_call(
      _atcombine_body,
      grid=(32,),
      in_specs=[pl.BlockSpec((2, 128, N), lambda i: (0, i, 0))],
      out_specs=pl.BlockSpec((128, N), lambda i: (i, 0)),
      out_shape=jax.ShapeDtypeStruct((N, N), jnp.bfloat16),
  )(at2)


_KP = 512


def _pow_dot(lhs_ref, atf_ref):
  def body(kp, acc):
    lsl = lhs_ref[:, pl.ds(kp * _KP, _KP)]
    panel = atf_ref[pl.ds(kp * _KP, _KP), :]
    return acc + lax.dot_general(lsl, panel, (((1,), (0,)), ((), ())),
                                 preferred_element_type=jnp.float32)

  init = jnp.zeros((_BR, N), jnp.float32)
  return lax.fori_loop(0, N // _KP, body, init)


def _dense_body(atf_ref, atb_ref, q_ref, k_ref, emb_ref, ex_ref, b2s_ref):
  a1 = atb_ref[...]
  p2 = _pow_dot(atb_ref, atf_ref)
  b2 = p2 > 0.0
  b2s_ref[...] = b2.astype(jnp.bfloat16)
  p3 = _pow_dot(b2s_ref, atf_ref)
  b3 = p3 > 0.0
  b1 = a1 > 0
  inv = jnp.float32(1.0 / (HD ** 0.5))
  for h in range(H):
    qh = q_ref[:, h * HD:(h + 1) * HD]
    kh = k_ref[:, h * HD:(h + 1) * HD]
    sc = lax.dot_general(qh, kh, (((1,), (1,)), ((), ())),
                         preferred_element_type=jnp.float32) * inv
    bias = jnp.where(b1, emb_ref[1, h],
                     jnp.where(b2, emb_ref[2, h],
                               jnp.where(b3, emb_ref[3, h], emb_ref[4, h])))
    ex_ref[h] = jnp.exp(sc + bias)


_BR = 64


def _dense_scores(atb, q, k, spd_emb):
  return pl.pallas_call(
      _dense_body,
      grid=(N // _BR,),
      in_specs=[
          pl.BlockSpec((N, N), lambda i: (0, 0)),
          pl.BlockSpec((_BR, N), lambda i: (i, 0)),
          pl.BlockSpec((_BR, C), lambda i: (i, 0)),
          pl.BlockSpec((N, C), lambda i: (0, 0)),
          pl.BlockSpec(memory_space=pltpu.SMEM),
      ],
      out_specs=pl.BlockSpec((H, _BR, N), lambda i: (0, i, 0)),
      out_shape=jax.ShapeDtypeStruct((H, N, N), jnp.float32),
      scratch_shapes=[pltpu.VMEM((_BR, N), jnp.bfloat16)],
      compiler_params=pltpu.CompilerParams(
          vmem_limit_bytes=60 * 1024 * 1024),
  )(atb, atb, q, k, spd_emb)


def _sc_edge_body(edge_ref, ex_ref, v_ref, seg_ref, agg_ref,
                  zbuf, zbuf1, sbuf, dbuf, gidx, sidx, vidx, aidx, exb, vrows,
                  vout, segtab, aggtab, gsem, vsem, ssem, asem):
  c = lax.axis_index("c")
  s = lax.axis_index("s")
  w = c * NS + s

  z16 = jnp.zeros((16,), jnp.float32)
  for jj in range(8):
    for rr in range(16):
      zbuf[rr, pl.ds(jj * 16, 16)] = z16
  _zero_fill(zbuf1, 2048)

  for r in range(16):
    pltpu.sync_copy(zbuf, aggtab.at[pl.ds(s * 256 + r * 16, 16)])

  @pl.when(s == 0)
  def _():
    for r in range(8):
      pltpu.sync_copy(zbuf1, segtab.at[pl.ds(r * 2048, 2048)])

  pltpu.sync_copy(edge_ref.at[0, pl.ds(w * EPW, EPW)], sbuf)
  pltpu.sync_copy(edge_ref.at[1, pl.ds(w * EPW, EPW)], dbuf)

  plsc.subcore_barrier()

  viota = lax.iota(jnp.int32, 16)

  @pl.loop(0, EPW // 256)
  def _(g):
    base = g * 256
    for r in range(2):
      for j in range(8):
        off = base + r * 128 + j * 16
        sv = sbuf[pl.ds(off, 16)]
        dv = dbuf[pl.ds(off, 16)]
        i0 = dv * N + sv
        s0 = dv * H
        for h in range(H):
          gidx[h * 2 + r, pl.ds(j * 16, 16)] = i0 + h * NN
          sidx[h * 2 + r, pl.ds(j * 16, 16)] = s0 + h
        vidx[r, pl.ds(j * 16, 16)] = sv
        aidx[r, pl.ds(j * 16, 16)] = dv
    gds = []
    for t in range(8):
      gds.append(pltpu.async_copy(ex_ref.at[gidx.at[t]],
                                  exb.at[pl.ds(t * 128, 128)], gsem))
    for r in range(2):
      gds.append(
          pltpu.async_copy(v_ref.at[vidx.at[r]],
                           vrows.at[pl.ds(r * 128, 128)], vsem))
    for d in gds:
      d.wait()
    sds = []
    for t in range(8):
      sds.append(
          pltpu.async_copy(exb.at[pl.ds(t * 128, 128)],
                           segtab.at[sidx.at[t]], ssem, add=True))
    @plsc.parallel_loop(0, 16, 1, unroll=4)
    def _(gg):
      rv = viota + gg * 16
      ebase = (gg // 8) * 128 + (gg % 8) * 16
      for h in range(H):
        wv = exb[pl.ds(h * 256 + ebase, 16)]
        for c in range(h * HD, (h + 1) * HD):
          cv = jnp.full((16,), c, jnp.int32)
          val = plsc.load_gather(vrows, [rv, cv])
          plsc.store_scatter(vout, [rv, cv], val * wv)
    for r in range(2):
      sds.append(
          pltpu.async_copy(vout.at[pl.ds(r * 128, 128)],
                           aggtab.at[aidx.at[r]], asem, add=True))
    for d in sds:
      d.wait()

  plsc.subcore_barrier()

  for r in range(16):
    pltpu.sync_copy(aggtab.at[pl.ds(s * 256 + r * 16, 16)],
                    agg_ref.at[pl.ds(c * N + s * 256 + r * 16, 16)])

  @pl.when(s == 0)
  def _():
    pltpu.sync_copy(segtab, seg_ref.at[pl.ds(c * N * H, N * H)])


def _sc_edge(edge_index, exf, v):
  return pl.kernel(
      _sc_edge_body,
      out_type=[
          jax.ShapeDtypeStruct((2 * N * H,), jnp.float32),
          jax.ShapeDtypeStruct((2 * N, C), jnp.float32),
      ],
      mesh=_MESH,
      scratch_types=[
          pltpu.VMEM((16, 128), jnp.float32),
          pltpu.VMEM((2048,), jnp.float32),
          pltpu.VMEM((EPW,), jnp.int32),
          pltpu.VMEM((EPW,), jnp.int32),
          pltpu.VMEM((8, 128), jnp.int32),
          pltpu.VMEM((8, 128), jnp.int32),
          pltpu.VMEM((2, 128), jnp.int32),
          pltpu.VMEM((2, 128), jnp.int32),
          pltpu.VMEM((1024,), jnp.float32),
          pltpu.VMEM((256, C), jnp.float32),
          pltpu.VMEM((256, C), jnp.float32),
          pltpu.VMEM_SHARED((N * H,), jnp.float32),
          pltpu.VMEM_SHARED((N, C), jnp.float32),
          pltpu.SemaphoreType.DMA,
          pltpu.SemaphoreType.DMA,
          pltpu.SemaphoreType.DMA,
          pltpu.SemaphoreType.DMA,
      ],
      compiler_params=_SC_PARAMS,
  )(edge_index, exf, v)


def _final_body(agg_ref, seg_ref, x_ref, deg_ref, g_ref, b_ref, y_ref):
  d2 = deg_ref[0] + deg_ref[1]
  dcol = jnp.sum(d2, axis=1, keepdims=True)
  segc = seg_ref[0] + seg_ref[1]
  segb = jnp.reshape(
      lax.broadcast_in_dim(segc, (N, H, HD), (0, 1)), (N, C))
  att = (agg_ref[0] + agg_ref[1]) / (segb + 1e-16)
  hh = att + x_ref[...] + dcol
  mu = jnp.mean(hh, axis=1, keepdims=True)
  dd = hh - mu
  var = jnp.mean(dd * dd, axis=1, keepdims=True)
  y_ref[...] = dd / jnp.sqrt(var + 1e-5) * g_ref[...] + b_ref[...]


def _final(agg, seg, x, degs, ln_g, ln_b):
  return pl.pallas_call(
      _final_body,
      out_shape=jax.ShapeDtypeStruct((N, C), jnp.float32),
  )(agg, seg, x, degs, ln_g.reshape(1, C), ln_b.reshape(1, C))


def kernel(x, edge_index, Wq, bq, Wk, bk, Wv, bv, spd_emb, ln_g, ln_b):
  at2_flat, degs_flat = _sc_build(edge_index)
  q, k, v = _qkv(x, Wq, bq, Wk, bk, Wv, bv)
  atb = _atcombine(at2_flat.reshape(2, N, N))
  ex = _dense_scores(atb, q, k, spd_emb)
  exf = ex.reshape(H * NN)
  seg, agg = _sc_edge(edge_index, exf, v)
  return _final(agg.reshape(2, N, C), seg.reshape(2, N, H), x,
                degs_flat.reshape(2, N, 2), ln_g, ln_b)
